# Initial kernel scaffold; baseline (speedup 1.0000x reference)
import jax
import jax.numpy as jnp
from jax.experimental import pallas as pl


def kernel(hidden, node_attention, query, rel_table, ws, b, out_w, out_b, W_con, b_con, W_uncon, b_uncon, selected_edges):
    raise NotImplementedError("write your pallas kernel here")


# speedup vs baseline: 11.4858x; 11.4858x over previous
import functools

import jax
import jax.numpy as jnp
from jax import lax
from jax.experimental import pallas as pl
from jax.experimental.pallas import tpu as pltpu
from jax.experimental.pallas import tpu_sc as plsc

NC, NS = 2, 16
NW = NC * NS
L = 16

NN = 50000
NP = 50048
PAD_SLOT = 50040
E = 800000
CHUNK = E // NW
CPAD = 25088
EP = NW * CPAD
NROW = EP // 128
SBLK = 512
NSB = CPAD // SBLK
NRL = 500
D = 32


def _mesh():
    return plsc.VectorSubcoreMesh(
        core_axis_name="c", subcore_axis_name="s", num_cores=NC, num_subcores=NS
    )


def _k1_body(hidden, vi1, vj1, rel1, rel_tab, coef, btbl, owtbl,
             numer_o, denom_o,
             P2, Q2, denom_loc, hi_rows, hj_rows,
             vi_sbf, vj_sbf, rel_sbf, numer_sbf,
             coef_v, btbl_v, owtbl_v, sem):
    wid = lax.axis_index("s") * NC + lax.axis_index("c")

    pltpu.sync_copy(coef, coef_v)
    pltpu.sync_copy(btbl, btbl_v)
    pltpu.sync_copy(owtbl, owtbl_v)
    pltpu.sync_copy(rel_tab, hi_rows.at[pl.ds(0, NRL)])

    a0 = coef_v[0, pl.ds(0, L)]
    a1 = coef_v[0, pl.ds(L, L)]
    b0 = coef_v[1, pl.ds(0, L)]
    b1 = coef_v[1, pl.ds(L, L)]
    c0 = coef_v[2, pl.ds(0, L)]
    c1 = coef_v[2, pl.ds(L, L)]
    d0 = coef_v[3, pl.ds(0, L)]
    d1 = coef_v[3, pl.ds(L, L)]

    def pq_body(r, _):
        r0 = hi_rows[r, pl.ds(0, L)]
        r1 = hi_rows[r, pl.ds(L, L)]
        P2[r, pl.ds(0, L)] = a0 + r0 * b0
        P2[r, pl.ds(L, L)] = a1 + r1 * b1
        Q2[r, pl.ds(0, L)] = c0 + r0 * d0
        Q2[r, pl.ds(L, L)] = c1 + r1 * d1
        return 0

    lax.fori_loop(0, NRL, pq_body, 0)

    zv = jnp.zeros((L,), jnp.float32)

    def zero_body(i, _):
        denom_loc[pl.ds(i * L, L)] = zv
        return 0

    lax.fori_loop(0, NP // L, zero_body, 0)

    iota = lax.iota(jnp.int32, L)

    def sb_body(sb, _):
        base = wid * CPAD + sb * SBLK
        pltpu.sync_copy(vi1.at[pl.ds(base, SBLK)], vi_sbf)
        pltpu.sync_copy(vj1.at[pl.ds(base, SBLK)], vj_sbf)
        pltpu.sync_copy(rel1.at[pl.ds(base, SBLK)], rel_sbf)
        cps = []
        for j in range(4):
            cps.append(pltpu.async_copy(
                hidden.at[vi_sbf.at[pl.ds(j * 128, 128)]],
                hi_rows.at[pl.ds(j * 128, 128)], sem))
            cps.append(pltpu.async_copy(
                hidden.at[vj_sbf.at[pl.ds(j * 128, 128)]],
                hj_rows.at[pl.ds(j * 128, 128)], sem))
        for cp in cps:
            cp.wait()

        def g_body(g, _):
            e16 = g * L
            vi_ids = vi_sbf[pl.ds(e16, L)]
            rel_ids = rel_sbf[pl.ds(e16, L)]
            erow = e16 + iota
            acc = jnp.zeros((L,), jnp.float32)
            for d in range(D):
                dd = jnp.full((L,), d, jnp.int32)
                hi_d = plsc.load_gather(hi_rows, [erow, dd])
                hj_d = plsc.load_gather(hj_rows, [erow, dd])
                p_d = plsc.load_gather(P2, [rel_ids, dd])
                q_d = plsc.load_gather(Q2, [rel_ids, dd])
                s = hi_d * p_d + hj_d * q_d + btbl_v[d]
                acc = acc + jnp.maximum(s, 0.0) * owtbl_v[d]
            numer = jnp.exp(acc)
            numer_sbf[pl.ds(e16, L)] = numer
            plsc.addupdate_scatter(denom_loc, [vi_ids], numer)
            return 0

        lax.fori_loop(0, SBLK // L, g_body, 0)
        pltpu.sync_copy(numer_sbf, numer_o.at[pl.ds(base, SBLK)])
        return 0

    lax.fori_loop(0, NSB, sb_body, 0)
    pltpu.sync_copy(denom_loc, denom_o.at[wid])


def _k2_body(vi1, vj1, numer1, att2, den2,
             out_o,
             ratio, out_loc, att_t, den_t, vi_sbf, vj_sbf, num_sbf, sem):
    wid = lax.axis_index("s") * NC + lax.axis_index("c")

    zv = jnp.zeros((L,), jnp.float32)

    def zero_body(i, _):
        out_loc[pl.ds(i * L, L)] = zv
        return 0

    lax.fori_loop(0, NP // L, zero_body, 0)

    def ratio_chunk(c, _):
        pltpu.sync_copy(att2.at[pl.ds(c * 23, 23)], att_t)
        pltpu.sync_copy(den2.at[pl.ds(c * 23, 23)], den_t)

        def rr_body(rr, _):
            off = (c * 23 + rr) * 128
            for k in range(8):
                a = att_t[rr, pl.ds(k * L, L)]
                dn = den_t[rr, pl.ds(k * L, L)]
                ratio[pl.ds(off + k * L, L)] = a / dn
            return 0

        lax.fori_loop(0, 23, rr_body, 0)
        return 0

    lax.fori_loop(0, 17, ratio_chunk, 0)

    def sb_body(sb, _):
        base = wid * CPAD + sb * SBLK
        pltpu.sync_copy(vi1.at[pl.ds(base, SBLK)], vi_sbf)
        pltpu.sync_copy(vj1.at[pl.ds(base, SBLK)], vj_sbf)
        pltpu.sync_copy(numer1.at[pl.ds(base, SBLK)], num_sbf)

        def g_body(g, _):
            e16 = g * L
            vi_ids = vi_sbf[pl.ds(e16, L)]
            vj_ids = vj_sbf[pl.ds(e16, L)]
            nmr = num_sbf[pl.ds(e16, L)]
            rat = plsc.load_gather(ratio, [vi_ids])
            plsc.addupdate_scatter(out_loc, [vj_ids], nmr * rat)
            return 0

        lax.fori_loop(0, SBLK // L, g_body, 0)
        return 0

    lax.fori_loop(0, NSB, sb_body, 0)
    pltpu.sync_copy(out_loc, out_o.at[wid])


@functools.partial(
    pl.kernel,
    out_type=(
        jax.ShapeDtypeStruct((EP,), jnp.float32),
        jax.ShapeDtypeStruct((NW, NP), jnp.float32),
    ),
    mesh=_mesh(),
    scratch_types=[
        pltpu.VMEM((NRL, D), jnp.float32),
        pltpu.VMEM((NRL, D), jnp.float32),
        pltpu.VMEM((NP,), jnp.float32),
        pltpu.VMEM((SBLK, D), jnp.float32),
        pltpu.VMEM((SBLK, D), jnp.float32),
        pltpu.VMEM((SBLK,), jnp.int32),
        pltpu.VMEM((SBLK,), jnp.int32),
        pltpu.VMEM((SBLK,), jnp.int32),
        pltpu.VMEM((SBLK,), jnp.float32),
        pltpu.VMEM((4, D), jnp.float32),
        pltpu.VMEM((D, L), jnp.float32),
        pltpu.VMEM((D, L), jnp.float32),
        pltpu.SemaphoreType.DMA,
    ],
    compiler_params=pltpu.CompilerParams(needs_layout_passes=False, use_tc_tiling_on_sc=False),
    name="attflow_k1",
)
def _k1(*args):
    _k1_body(*args)


@functools.partial(
    pl.kernel,
    out_type=jax.ShapeDtypeStruct((NW, NP), jnp.float32),
    mesh=_mesh(),
    scratch_types=[
        pltpu.VMEM((NP,), jnp.float32),
        pltpu.VMEM((NP,), jnp.float32),
        pltpu.VMEM((23, 128), jnp.float32),
        pltpu.VMEM((23, 128), jnp.float32),
        pltpu.VMEM((SBLK,), jnp.int32),
        pltpu.VMEM((SBLK,), jnp.int32),
        pltpu.VMEM((SBLK,), jnp.float32),
        pltpu.SemaphoreType.DMA,
    ],
    compiler_params=pltpu.CompilerParams(needs_layout_passes=False, use_tc_tiling_on_sc=False),
    name="attflow_k2",
)
def _k2(*args):
    _k2_body(*args)


def kernel(hidden, node_attention, query, rel_table, ws, b, out_w, out_b,
           W_con, b_con, W_uncon, b_uncon, selected_edges):
    f32 = jnp.float32
    con = jnp.tanh(query @ W_con + b_con)
    uncon = jnp.tanh(query @ W_uncon + b_uncon)
    A = con * ws[0] + uncon * ws[2]
    B = con * ws[1] + uncon * ws[3]
    C = con * ws[4] + uncon * ws[6]
    Dv = con * ws[5] + uncon * ws[7]
    coef = jnp.stack([A, B, C, Dv]).astype(f32)
    btbl = jnp.broadcast_to(b[:, None], (D, L)).astype(f32)
    owtbl = jnp.broadcast_to(out_w[:, None], (D, L)).astype(f32)

    se = selected_edges
    vi = se[:, 1]
    vj = se[:, 2]
    rel = se[:, 3]

    def pad_edges(x, fill):
        xr = x.reshape(NW, CHUNK)
        xr = jnp.pad(xr, ((0, 0), (0, CPAD - CHUNK)), constant_values=fill)
        return xr.reshape(EP)

    vi1 = pad_edges(vi, PAD_SLOT)
    vj1 = pad_edges(vj, PAD_SLOT)
    rel1 = pad_edges(rel, 0)

    hidden_p = jnp.pad(hidden, ((0, NP - NN), (0, 0))).astype(f32)
    att_p = jnp.pad(node_attention, (0, NP - NN)).astype(f32)

    numer, denom_part = _k1(hidden_p, vi1, vj1, rel1,
                            rel_table.astype(f32), coef, btbl, owtbl)
    denom = jnp.sum(denom_part, axis=0).reshape(NP // 128, 128)
    att2 = att_p.reshape(NP // 128, 128)

    out_part = _k2(vi1, vj1, numer, att2, denom)
    return jnp.sum(out_part, axis=0)[:NN].reshape(1, NN)


# speedup vs baseline: 28.4153x; 2.4739x over previous
import functools

import jax
import jax.numpy as jnp
from jax import lax
from jax.experimental import pallas as pl
from jax.experimental.pallas import tpu as pltpu
from jax.experimental.pallas import tpu_sc as plsc

NC, NS = 2, 16
NW = NC * NS
L = 16

NN = 50000
NP = 50048
PAD_SLOT = 50040
E = 800000
CHUNK = E // NW
CPAD = 25088
EP = NW * CPAD
NROW = EP // 128
SBLK = 512
NSB = CPAD // SBLK
NRL = 500
D = 32


def _mesh():
    return plsc.VectorSubcoreMesh(
        core_axis_name="c", subcore_axis_name="s", num_cores=NC, num_subcores=NS
    )


def _k1_body(hidden, vi1, vj1, rel1, rel_tab, coef, bvec, owvec,
             numer_o, denom_o,
             P2, Q2, denom_loc, hi_rows, hj_rows,
             vi_sbf, vj_sbf, rel_sbf, numer_sbf,
             coef_v, bvec_v, owvec_v, sem):
    wid = lax.axis_index("s") * NC + lax.axis_index("c")

    pltpu.sync_copy(coef, coef_v)
    pltpu.sync_copy(bvec, bvec_v)
    pltpu.sync_copy(owvec, owvec_v)
    pltpu.sync_copy(rel_tab, hi_rows.at[pl.ds(0, NRL)])

    a0 = coef_v[0, pl.ds(0, L)]
    a1 = coef_v[0, pl.ds(L, L)]
    b0 = coef_v[1, pl.ds(0, L)]
    b1 = coef_v[1, pl.ds(L, L)]
    c0 = coef_v[2, pl.ds(0, L)]
    c1 = coef_v[2, pl.ds(L, L)]
    d0 = coef_v[3, pl.ds(0, L)]
    d1 = coef_v[3, pl.ds(L, L)]

    def pq_body(r, _):
        r0 = hi_rows[r, pl.ds(0, L)]
        r1 = hi_rows[r, pl.ds(L, L)]
        P2[r, pl.ds(0, L)] = a0 + r0 * b0
        P2[r, pl.ds(L, L)] = a1 + r1 * b1
        Q2[r, pl.ds(0, L)] = c0 + r0 * d0
        Q2[r, pl.ds(L, L)] = c1 + r1 * d1
        return 0

    lax.fori_loop(0, NRL, pq_body, 0)

    zv = jnp.zeros((L,), jnp.float32)

    def zero_body(i, _):
        denom_loc[pl.ds(i * L, L)] = zv
        return 0

    lax.fori_loop(0, NP // L, zero_body, 0)

    iota = lax.iota(jnp.int32, L)

    def sb_body(sb, _):
        base = wid * CPAD + sb * SBLK
        pltpu.sync_copy(vi1.at[pl.ds(base, SBLK)], vi_sbf)
        pltpu.sync_copy(vj1.at[pl.ds(base, SBLK)], vj_sbf)
        pltpu.sync_copy(rel1.at[pl.ds(base, SBLK)], rel_sbf)
        cps = []
        for j in range(4):
            cps.append(pltpu.async_copy(
                hidden.at[vi_sbf.at[pl.ds(j * 128, 128)]],
                hi_rows.at[pl.ds(j * 128, 128)], sem))
            cps.append(pltpu.async_copy(
                hidden.at[vj_sbf.at[pl.ds(j * 128, 128)]],
                hj_rows.at[pl.ds(j * 128, 128)], sem))
        for cp in cps:
            cp.wait()

        def g_body(g, _):
            e16 = g * L
            vi_ids = vi_sbf[pl.ds(e16, L)]
            rel_ids = rel_sbf[pl.ds(e16, L)]
            erow = e16 + iota
            acc = jnp.zeros((L,), jnp.float32)
            for d in range(D):
                dd = jnp.bitwise_and(iota + d, D - 1)
                hi_d = plsc.load_gather(hi_rows, [erow, dd])
                hj_d = plsc.load_gather(hj_rows, [erow, dd])
                p_d = plsc.load_gather(P2, [rel_ids, dd])
                q_d = plsc.load_gather(Q2, [rel_ids, dd])
                b_d = plsc.load_gather(bvec_v, [dd])
                ow_d = plsc.load_gather(owvec_v, [dd])
                s = hi_d * p_d + hj_d * q_d + b_d
                acc = acc + jnp.maximum(s, 0.0) * ow_d
            numer = jnp.exp(acc)
            numer_sbf[pl.ds(e16, L)] = numer
            plsc.addupdate_scatter(denom_loc, [vi_ids], numer)
            return 0

        lax.fori_loop(0, SBLK // L, g_body, 0)
        pltpu.sync_copy(numer_sbf, numer_o.at[pl.ds(base, SBLK)])
        return 0

    lax.fori_loop(0, NSB, sb_body, 0)
    pltpu.sync_copy(denom_loc, denom_o.at[wid])


def _k2_body(vi1, vj1, numer1, att2, den2,
             out_o,
             ratio, out_loc, att_t, den_t, vi_sbf, vj_sbf, num_sbf, sem):
    wid = lax.axis_index("s") * NC + lax.axis_index("c")

    zv = jnp.zeros((L,), jnp.float32)

    def zero_body(i, _):
        out_loc[pl.ds(i * L, L)] = zv
        return 0

    lax.fori_loop(0, NP // L, zero_body, 0)

    def ratio_chunk(c, _):
        pltpu.sync_copy(att2.at[pl.ds(c * 23, 23)], att_t)
        pltpu.sync_copy(den2.at[pl.ds(c * 23, 23)], den_t)

        def rr_body(rr, _):
            off = (c * 23 + rr) * 128
            for k in range(8):
                a = att_t[rr, pl.ds(k * L, L)]
                dn = den_t[rr, pl.ds(k * L, L)]
                ratio[pl.ds(off + k * L, L)] = a / dn
            return 0

        lax.fori_loop(0, 23, rr_body, 0)
        return 0

    lax.fori_loop(0, 17, ratio_chunk, 0)

    def sb_body(sb, _):
        base = wid * CPAD + sb * SBLK
        pltpu.sync_copy(vi1.at[pl.ds(base, SBLK)], vi_sbf)
        pltpu.sync_copy(vj1.at[pl.ds(base, SBLK)], vj_sbf)
        pltpu.sync_copy(numer1.at[pl.ds(base, SBLK)], num_sbf)

        def g_body(g, _):
            e16 = g * L
            vi_ids = vi_sbf[pl.ds(e16, L)]
            vj_ids = vj_sbf[pl.ds(e16, L)]
            nmr = num_sbf[pl.ds(e16, L)]
            rat = plsc.load_gather(ratio, [vi_ids])
            plsc.addupdate_scatter(out_loc, [vj_ids], nmr * rat)
            return 0

        lax.fori_loop(0, SBLK // L, g_body, 0)
        return 0

    lax.fori_loop(0, NSB, sb_body, 0)
    pltpu.sync_copy(out_loc, out_o.at[wid])


@functools.partial(
    pl.kernel,
    out_type=(
        jax.ShapeDtypeStruct((EP,), jnp.float32),
        jax.ShapeDtypeStruct((NW, NP), jnp.float32),
    ),
    mesh=_mesh(),
    scratch_types=[
        pltpu.VMEM((NRL, D), jnp.float32),
        pltpu.VMEM((NRL, D), jnp.float32),
        pltpu.VMEM((NP,), jnp.float32),
        pltpu.VMEM((SBLK, D), jnp.float32),
        pltpu.VMEM((SBLK, D), jnp.float32),
        pltpu.VMEM((SBLK,), jnp.int32),
        pltpu.VMEM((SBLK,), jnp.int32),
        pltpu.VMEM((SBLK,), jnp.int32),
        pltpu.VMEM((SBLK,), jnp.float32),
        pltpu.VMEM((4, D), jnp.float32),
        pltpu.VMEM((D,), jnp.float32),
        pltpu.VMEM((D,), jnp.float32),
        pltpu.SemaphoreType.DMA,
    ],
    compiler_params=pltpu.CompilerParams(needs_layout_passes=False, use_tc_tiling_on_sc=False),
    name="attflow_k1",
)
def _k1(*args):
    _k1_body(*args)


@functools.partial(
    pl.kernel,
    out_type=jax.ShapeDtypeStruct((NW, NP), jnp.float32),
    mesh=_mesh(),
    scratch_types=[
        pltpu.VMEM((NP,), jnp.float32),
        pltpu.VMEM((NP,), jnp.float32),
        pltpu.VMEM((23, 128), jnp.float32),
        pltpu.VMEM((23, 128), jnp.float32),
        pltpu.VMEM((SBLK,), jnp.int32),
        pltpu.VMEM((SBLK,), jnp.int32),
        pltpu.VMEM((SBLK,), jnp.float32),
        pltpu.SemaphoreType.DMA,
    ],
    compiler_params=pltpu.CompilerParams(needs_layout_passes=False, use_tc_tiling_on_sc=False),
    name="attflow_k2",
)
def _k2(*args):
    _k2_body(*args)


def kernel(hidden, node_attention, query, rel_table, ws, b, out_w, out_b,
           W_con, b_con, W_uncon, b_uncon, selected_edges):
    f32 = jnp.float32
    con = jnp.tanh(query @ W_con + b_con)
    uncon = jnp.tanh(query @ W_uncon + b_uncon)
    A = con * ws[0] + uncon * ws[2]
    B = con * ws[1] + uncon * ws[3]
    C = con * ws[4] + uncon * ws[6]
    Dv = con * ws[5] + uncon * ws[7]
    coef = jnp.stack([A, B, C, Dv]).astype(f32)
    bvec = b.astype(f32)
    owvec = out_w.astype(f32)

    se = selected_edges
    vi = se[:, 1]
    vj = se[:, 2]
    rel = se[:, 3]

    def pad_edges(x, fill):
        xr = x.reshape(NW, CHUNK)
        xr = jnp.pad(xr, ((0, 0), (0, CPAD - CHUNK)), constant_values=fill)
        return xr.reshape(EP)

    vi1 = pad_edges(vi, PAD_SLOT)
    vj1 = pad_edges(vj, PAD_SLOT)
    rel1 = pad_edges(rel, 0)

    hidden_p = jnp.pad(hidden, ((0, NP - NN), (0, 0))).astype(f32)
    att_p = jnp.pad(node_attention, (0, NP - NN)).astype(f32)

    numer, denom_part = _k1(hidden_p, vi1, vj1, rel1,
                            rel_table.astype(f32), coef, bvec, owvec)
    denom = jnp.sum(denom_part, axis=0).reshape(NP // 128, 128)
    att2 = att_p.reshape(NP // 128, 128)

    out_part = _k2(vi1, vj1, numer, att2, denom)
    return jnp.sum(out_part, axis=0)[:NN].reshape(1, NN)
